# baseline (speedup 1.0000x reference)
import functools

import jax
import jax.numpy as jnp
from jax import lax
from jax.experimental import pallas as pl
from jax.experimental.pallas import tpu as pltpu
from jax.experimental.pallas import tpu_sc as plsc

_B, _N, _K = 64, 196, 4096
_TOK = _B * _N
_NW = 32
_RW = 400
_TOK_PAD = _NW * _RW
_NCH = _RW // 16
_TROWS = _N * 8 * 32 * 8

_Q0 = -1.7306266776741692
_Q1 = 2.792243060183196
_Q2 = -1.4424703198838515
_Q3 = 0.43585782883908253
_Q4 = -0.05486231128932941
_LN2 = 0.6931471805599453


def _lnvec(x):
    bits = plsc.bitcast(x, jnp.int32)
    e = lax.shift_right_logical(bits, 23) - 127
    m = plsc.bitcast((bits & 0x007FFFFF) | 0x3F800000, jnp.float32)
    q = (((_Q4 * m + _Q3) * m + _Q2) * m + _Q1) * m + _Q0
    lnx = e.astype(jnp.float32) * _LN2 + q
    return jnp.where(x == 0.0, -jnp.inf, lnx)


def _sc_body(mask_hbm, ps_hbm, pt_hbm, out_hbm,
             mask_v, tok_v, idx0, idx1, ps0, ps1, pt0, pt1, accv, csbuf,
             sps0, sps1, spt0, spt1):
    w = lax.axis_index("s") * 2 + lax.axis_index("c")
    base_tok = w * _RW

    pltpu.sync_copy(mask_hbm.at[pl.ds(base_tok, _RW)], mask_v)

    zeros16 = jnp.zeros((16,), jnp.int32)
    for j in range(8):
        idx0[pl.ds(j * 16, 16)] = zeros16
        idx1[pl.ds(j * 16, 16)] = zeros16

    for j in range(_NCH):
        tok_v[j] = zeros16

    iota16 = lax.iota(jnp.int32, 16)

    def _prefix16(v):
        cur = v
        for d in (1, 2, 4, 8):
            csbuf[...] = cur
            sh = plsc.load_gather(csbuf, [jnp.maximum(iota16 - d, 0)])
            cur = cur + jnp.where(iota16 >= d, sh, 0)
        return cur

    def _compact(j, cum):
        v = mask_v[pl.ds(j * 16, 16)]
        ids = base_tok + j * 16 + iota16
        cs = _prefix16(v)
        pos = cum + cs - v
        plsc.store_scatter(tok_v, [pos >> 4, pos & 15], ids, mask=v > 0)
        return cum + lax.squeeze(lax.slice(cs, (15,), (16,)), (0,))

    c_w = jnp.int32(0)
    for j in range(_NCH):
        c_w = _compact(j, c_w)

    n_chunks = (c_w + 15) >> 4
    total_units = n_chunks * 4

    bufs = ((idx0, ps0, pt0, sps0, spt0), (idx1, ps1, pt1, sps1, spt1))

    def _build_idx(u, idx_ref):
        g = u // 4
        kq = u % 4
        tv = tok_v[g]
        n = lax.shift_right_logical(tv, 6)
        b = tv & 63
        base0 = n * 2048 + (b >> 3) * 256 + (b & 7) + kq * 64
        for j in range(8):
            idx_ref[pl.ds(j * 16, 16)] = base0 + j * 8

    def _fire(u, slot):
        idx_ref, ps_r, pt_r, sps, spt = bufs[slot]
        _build_idx(u, idx_ref)
        pltpu.async_copy(ps_hbm.at[idx_ref], ps_r, sps)
        pltpu.async_copy(pt_hbm.at[idx_ref], pt_r, spt)

    def _wait(slot):
        idx_ref, ps_r, pt_r, sps, spt = bufs[slot]
        pltpu.make_async_copy(ps_hbm.at[idx_ref], ps_r, sps).wait()
        pltpu.make_async_copy(pt_hbm.at[idx_ref], pt_r, spt).wait()

    def _compute(u, slot, acc):
        idx_ref, ps_r, pt_r, sps, spt = bufs[slot]
        g = u // 4
        valid = jnp.minimum(c_w - g * 16, 16)

        def _tok(t, a):
            for j in range(8):
                row = j * 16 + t
                for c in range(8):
                    x = ps_r[row, pl.ds(c * 16, 16)]
                    y = pt_r[row, pl.ds(c * 16, 16)]
                    a = a + y * _lnvec(x)
            return a

        return lax.fori_loop(0, valid, _tok, acc)

    @pl.when(total_units > 0)
    def _():
        _fire(jnp.int32(0), 0)

    def _pair(u2, acc):
        for b in (0, 1):
            u = u2 * 2 + b

            def _do(a, u=u, b=b):
                @pl.when(u + 1 < total_units)
                def _():
                    _fire(u + 1, 1 - b)

                _wait(b)
                return _compute(u, b, a)

            acc = lax.cond(u < total_units, _do, lambda a: a, acc)
        return acc

    n_pairs = (total_units + 1) // 2
    acc = lax.fori_loop(0, n_pairs, _pair, jnp.zeros((16,), jnp.float32))

    accv[...] = acc
    pltpu.sync_copy(accv, out_hbm.at[w])


def kernel(ps, pt, bool_masked_pos):
    def table(x):
        xt = jnp.transpose(x, (1, 0, 2))
        x5 = xt.reshape(_N, 8, 8, 32, 128)
        return jnp.transpose(x5, (0, 1, 3, 2, 4)).reshape(_TROWS, 128)

    ps2d = table(ps)
    pt2d = table(pt)

    maskT = jnp.transpose(bool_masked_pos, (1, 0)).reshape(_TOK)
    mask_pad = jnp.zeros((_TOK_PAD,), jnp.int32).at[:_TOK].set(
        maskT.astype(jnp.int32))

    mesh = plsc.VectorSubcoreMesh(core_axis_name="c", subcore_axis_name="s")
    sc = functools.partial(
        pl.kernel,
        mesh=mesh,
        compiler_params=pltpu.CompilerParams(needs_layout_passes=False),
        out_type=jax.ShapeDtypeStruct((_NW, 16), jnp.float32),
        scratch_types=[
            pltpu.VMEM((_RW,), jnp.int32),
            pltpu.VMEM((_NCH, 16), jnp.int32),
            pltpu.VMEM((128,), jnp.int32),
            pltpu.VMEM((128,), jnp.int32),
            pltpu.VMEM((128, 128), jnp.float32),
            pltpu.VMEM((128, 128), jnp.float32),
            pltpu.VMEM((128, 128), jnp.float32),
            pltpu.VMEM((128, 128), jnp.float32),
            pltpu.VMEM((16,), jnp.float32),
            pltpu.VMEM((16,), jnp.int32),
            pltpu.SemaphoreType.DMA,
            pltpu.SemaphoreType.DMA,
            pltpu.SemaphoreType.DMA,
            pltpu.SemaphoreType.DMA,
        ],
    )(_sc_body)

    partial = sc(mask_pad, ps2d, pt2d)
    cnt = jnp.sum(bool_masked_pos.astype(jnp.float32))
    return -jnp.sum(partial) / cnt


# speedup vs baseline: 1.0458x; 1.0458x over previous
import functools

import jax
import jax.numpy as jnp
from jax import lax
from jax.experimental import pallas as pl
from jax.experimental.pallas import tpu as pltpu
from jax.experimental.pallas import tpu_sc as plsc

_B, _N, _K = 64, 196, 4096
_TOK = _B * _N
_NW = 32
_RW = 400
_TOK_PAD = _NW * _RW
_NCH = _RW // 16
_TROWS = _N * 8 * 32 * 8

_LN2 = 0.6931471805599453
_Q0 = -1.7306266776741692 - 127.0 * _LN2
_Q1 = 2.792243060183196
_Q2 = -1.4424703198838515
_Q3 = 0.43585782883908253
_Q4 = -0.05486231128932941


def _lnvec(x):
    bits = plsc.bitcast(x, jnp.int32)
    e = lax.shift_right_logical(bits, 23)
    m = plsc.bitcast((bits & 0x007FFFFF) | 0x3F800000, jnp.float32)
    q = (((_Q4 * m + _Q3) * m + _Q2) * m + _Q1) * m + _Q0
    lnx = e.astype(jnp.float32) * _LN2 + q
    return jnp.where(x == 0.0, -jnp.inf, lnx)


def _sc_body(mask_hbm, ps_hbm, pt_hbm, out_hbm,
             mask_v, tok_v, idx0, idx1, ps0, ps1, pt0, pt1, accv, csbuf,
             sps0, sps1, spt0, spt1):
    w = lax.axis_index("s") * 2 + lax.axis_index("c")
    base_tok = w * _RW

    pltpu.sync_copy(mask_hbm.at[pl.ds(base_tok, _RW)], mask_v)

    zeros16 = jnp.zeros((16,), jnp.int32)
    for j in range(8):
        idx0[pl.ds(j * 16, 16)] = zeros16
        idx1[pl.ds(j * 16, 16)] = zeros16

    for j in range(_NCH):
        tok_v[j] = zeros16

    iota16 = lax.iota(jnp.int32, 16)

    def _prefix16(v):
        cur = v
        for d in (1, 2, 4, 8):
            csbuf[...] = cur
            sh = plsc.load_gather(csbuf, [jnp.maximum(iota16 - d, 0)])
            cur = cur + jnp.where(iota16 >= d, sh, 0)
        return cur

    def _compact(j, cum):
        v = mask_v[pl.ds(j * 16, 16)]
        ids = base_tok + j * 16 + iota16
        cs = _prefix16(v)
        pos = cum + cs - v
        plsc.store_scatter(tok_v, [pos >> 4, pos & 15], ids, mask=v > 0)
        return cum + lax.squeeze(lax.slice(cs, (15,), (16,)), (0,))

    c_w = jnp.int32(0)
    for j in range(_NCH):
        c_w = _compact(j, c_w)

    n_chunks = (c_w + 15) >> 4
    total_units = n_chunks * 4

    bufs = ((idx0, ps0, pt0, sps0, spt0), (idx1, ps1, pt1, sps1, spt1))

    def _build_idx(u, idx_ref):
        g = u // 4
        kq = u % 4
        tv = tok_v[g]
        n = lax.shift_right_logical(tv, 6)
        b = tv & 63
        base0 = n * 2048 + (b >> 3) * 256 + (b & 7) + kq * 64
        for j in range(8):
            idx_ref[pl.ds(j * 16, 16)] = base0 + j * 8

    def _fire(u, slot):
        idx_ref, ps_r, pt_r, sps, spt = bufs[slot]
        _build_idx(u, idx_ref)
        pltpu.async_copy(ps_hbm.at[idx_ref], ps_r, sps)
        pltpu.async_copy(pt_hbm.at[idx_ref], pt_r, spt)

    def _wait(slot):
        idx_ref, ps_r, pt_r, sps, spt = bufs[slot]
        pltpu.make_async_copy(ps_hbm.at[idx_ref], ps_r, sps).wait()
        pltpu.make_async_copy(pt_hbm.at[idx_ref], pt_r, spt).wait()

    def _compute(u, slot, accs):
        idx_ref, ps_r, pt_r, sps, spt = bufs[slot]
        g = u // 4
        valid = jnp.minimum(c_w - g * 16, 16)

        def _tok(t, accs):
            a = list(accs)
            for j in range(8):
                row = j * 16 + t
                for c in range(8):
                    x = ps_r[row, pl.ds(c * 16, 16)]
                    y = pt_r[row, pl.ds(c * 16, 16)]
                    a[c] = a[c] + y * _lnvec(x)
            return tuple(a)

        return lax.fori_loop(0, valid, _tok, accs)

    @pl.when(total_units > 0)
    def _():
        _fire(jnp.int32(0), 0)

    def _pair(u2, accs):
        for b in (0, 1):
            u = u2 * 2 + b

            def _do(a, u=u, b=b):
                @pl.when(u + 1 < total_units)
                def _():
                    _fire(u + 1, 1 - b)

                _wait(b)
                return _compute(u, b, a)

            accs = lax.cond(u < total_units, _do, lambda a: a, accs)
        return accs

    n_pairs = (total_units + 1) // 2
    accs0 = tuple(jnp.zeros((16,), jnp.float32) for _ in range(8))
    accs = lax.fori_loop(0, n_pairs, _pair, accs0)

    acc = accs[0]
    for a in accs[1:]:
        acc = acc + a
    accv[...] = acc
    pltpu.sync_copy(accv, out_hbm.at[w])


def kernel(ps, pt, bool_masked_pos):
    def table(x):
        xt = jnp.transpose(x, (1, 0, 2))
        x5 = xt.reshape(_N, 8, 8, 32, 128)
        return jnp.transpose(x5, (0, 1, 3, 2, 4)).reshape(_TROWS, 128)

    ps2d = table(ps)
    pt2d = table(pt)

    maskT = jnp.transpose(bool_masked_pos, (1, 0)).reshape(_TOK)
    mask_pad = jnp.zeros((_TOK_PAD,), jnp.int32).at[:_TOK].set(
        maskT.astype(jnp.int32))

    mesh = plsc.VectorSubcoreMesh(core_axis_name="c", subcore_axis_name="s")
    sc = functools.partial(
        pl.kernel,
        mesh=mesh,
        compiler_params=pltpu.CompilerParams(needs_layout_passes=False),
        out_type=jax.ShapeDtypeStruct((_NW, 16), jnp.float32),
        scratch_types=[
            pltpu.VMEM((_RW,), jnp.int32),
            pltpu.VMEM((_NCH, 16), jnp.int32),
            pltpu.VMEM((128,), jnp.int32),
            pltpu.VMEM((128,), jnp.int32),
            pltpu.VMEM((128, 128), jnp.float32),
            pltpu.VMEM((128, 128), jnp.float32),
            pltpu.VMEM((128, 128), jnp.float32),
            pltpu.VMEM((128, 128), jnp.float32),
            pltpu.VMEM((16,), jnp.float32),
            pltpu.VMEM((16,), jnp.int32),
            pltpu.SemaphoreType.DMA,
            pltpu.SemaphoreType.DMA,
            pltpu.SemaphoreType.DMA,
            pltpu.SemaphoreType.DMA,
        ],
    )(_sc_body)

    partial = sc(mask_pad, ps2d, pt2d)
    cnt = jnp.sum(bool_masked_pos.astype(jnp.float32))
    return -jnp.sum(partial) / cnt
